# baseline (speedup 1.0000x reference)
import functools

import jax
import jax.numpy as jnp
from jax import lax
from jax.experimental import pallas as pl
from jax.experimental.pallas import tpu as pltpu
from jax.experimental.pallas import tpu_sc as plsc

B = 16384
D = 64
NC = 2
NS = 16
NW = NC * NS
BPW = B // NW
C = 128

NE = 1000000
BLKC = 2048
KFULL = NE // BLKC
NTAIL = NE - KFULL * BLKC
NPAIR = KFULL * 8
PPT = NPAIR // NW
NROW = (KFULL + 1) * (BLKC // 2)
DUMP = NROW - 128


def _repack_body(ent_hbm, out_hbm, ina, inb, gbufs, tl64, insems, outsems):
    wid = lax.axis_index("s") * NC + lax.axis_index("c")
    lane = lax.iota(jnp.int32, 16)
    rowv = [lane + 16 * q for q in range(4)]

    def fire(i, b):
        i = jnp.minimum(i, PPT - 1)
        pg = wid + 32 * i
        k = pg >> 3
        m = pg & 7
        c0 = pl.multiple_of((16 * k + m) * 128, 128)
        c1 = pl.multiple_of((16 * k + m + 8) * 128, 128)
        pltpu.async_copy(ent_hbm.at[:, pl.ds(c0, 128)], ina[b], insems[b])
        pltpu.async_copy(ent_hbm.at[:, pl.ds(c1, 128)], inb[b], insems[b])

    def drain_in(b):
        pltpu.make_async_copy(ent_hbm.at[:, pl.ds(0, 128)], ina[b],
                              insems[b]).wait()
        pltpu.make_async_copy(ent_hbm.at[:, pl.ds(0, 128)], inb[b],
                              insems[b]).wait()

    def drain_out(b):
        pltpu.make_async_copy(out_hbm.at[pl.ds(0, 128), :], gbufs[b],
                              outsems[b]).wait()

    def transpose(buf, g, hc):
        def tl(l4, _):
            for u in range(4):
                l = l4 * 4 + u
                colv = lane * 0 + l
                for q in range(4):
                    v = plsc.load_gather(buf, [rowv[q], colv])
                    g[l, pl.ds(hc * 64 + q * 16, 16)] = v
            return _

        lax.fori_loop(0, 32, tl, None)

    fire(0, 0)
    for b in range(2):
        pltpu.async_copy(gbufs[b], out_hbm.at[pl.ds(DUMP, 128), :],
                         outsems[b])

    def step(h, _):
        for b in range(2):
            i = h * 2 + b
            fire(i + 1, 1 - b)
            drain_in(b)
            drain_out(b)
            g = gbufs[b]
            transpose(ina[b], g, 0)
            transpose(inb[b], g, 1)
            pg = wid + 32 * i
            r0 = pl.multiple_of(((pg >> 3) * 1024 + (pg & 7) * 128), 8)
            pltpu.async_copy(g, out_hbm.at[pl.ds(r0, 128), :], outsems[b])
        return _

    lax.fori_loop(0, PPT // 2, step, None)
    drain_in(0)
    drain_out(0)
    drain_out(1)

    @pl.when(wid == 0)
    def _tail():
        for off in (0, 128, 256, 384, 512):
            last = off == 512
            g = gbufs[0]
            if last:
                pltpu.sync_copy(
                    ent_hbm.at[:, pl.ds(KFULL * BLKC + off, 64)], tl64)
            else:
                pltpu.sync_copy(
                    ent_hbm.at[:, pl.ds(KFULL * BLKC + off, 128)], ina[0])
            src = tl64 if last else ina[0]
            nl4 = 16 if last else 32

            def tl(l4, _):
                for u in range(4):
                    l = l4 * 4 + u
                    colv = lane * 0 + l
                    for q in range(4):
                        v = plsc.load_gather(src, [rowv[q], colv])
                        g[l, pl.ds(q * 16, 16)] = v
                return _

            lax.fori_loop(0, nl4, tl, None)
            pltpu.sync_copy(g, out_hbm.at[pl.ds(KFULL * 1024 + off,
                                                128), :])


def _repack_entity_table(ent_t):
    mesh = plsc.VectorSubcoreMesh(core_axis_name="c", subcore_axis_name="s")
    k = functools.partial(
        pl.kernel,
        mesh=mesh,
        compiler_params=pltpu.CompilerParams(needs_layout_passes=False),
        out_type=jax.ShapeDtypeStruct((NROW, 128), jnp.float32),
        scratch_types=[
            [pltpu.VMEM((D, 128), jnp.float32) for _ in range(2)],
            [pltpu.VMEM((D, 128), jnp.float32) for _ in range(2)],
            [pltpu.VMEM((128, 128), jnp.float32) for _ in range(2)],
            pltpu.VMEM((D, 64), jnp.float32),
            [pltpu.SemaphoreType.DMA for _ in range(2)],
            [pltpu.SemaphoreType.DMA for _ in range(2)],
        ],
    )(_repack_body)
    return k(ent_t)


def _tec_body(head_hbm, rel_hbm, tail_hbm, ent_hbm, relt_hbm, nrm_hbm,
              out_hbm, hidx, tidx, ridx, hoff, toff, roff, hrows, trows,
              rrows, wrows, oscr, sem):
    wid = lax.axis_index("s") * NC + lax.axis_index("c")
    base = wid * BPW

    pltpu.sync_copy(head_hbm.at[pl.ds(base, BPW)], hidx)
    pltpu.sync_copy(tail_hbm.at[pl.ds(base, BPW)], tidx)
    pltpu.sync_copy(rel_hbm.at[pl.ds(base, BPW)], ridx)

    sh = BLKC.bit_length() - 1
    hmask = BLKC // 2 - 1

    def split(g, _):
        hv = hidx[pl.ds(g * 16, 16)]
        tv = tidx[pl.ds(g * 16, 16)]
        rv = ridx[pl.ds(g * 16, 16)]
        hoff[pl.ds(g * 16, 16)] = ((hv >> (sh - 1)) & 1) << 6
        toff[pl.ds(g * 16, 16)] = ((tv >> (sh - 1)) & 1) << 6
        roff[pl.ds(g * 16, 16)] = (rv & 1) << 6
        hidx[pl.ds(g * 16, 16)] = ((hv >> sh) << (sh - 1)) | (hv & hmask)
        tidx[pl.ds(g * 16, 16)] = ((tv >> sh) << (sh - 1)) | (tv & hmask)
        ridx[pl.ds(g * 16, 16)] = rv >> 1
        return _

    lax.fori_loop(0, BPW // 16, split, None)

    lane = lax.iota(jnp.int32, 16)

    def chunk(c, carry0):
        off = c * C
        cph = pltpu.async_copy(ent_hbm.at[hidx.at[pl.ds(off, C)]], hrows,
                               sem)
        cpt = pltpu.async_copy(ent_hbm.at[tidx.at[pl.ds(off, C)]], trows,
                               sem)
        cpr = pltpu.async_copy(relt_hbm.at[ridx.at[pl.ds(off, C)]], rrows,
                               sem)
        cpw = pltpu.async_copy(nrm_hbm.at[ridx.at[pl.ds(off, C)]], wrows,
                               sem)
        cph.wait()
        cpt.wait()
        cpr.wait()
        cpw.wait()

        def group(g, carry):
            acc = jnp.zeros((16,), jnp.float32)
            phv = hoff[pl.ds(off + g * 16, 16)]
            ptv = toff[pl.ds(off + g * 16, 16)]
            prv = roff[pl.ds(off + g * 16, 16)]
            for j in range(16):
                e = g * 16 + j
                ph = phv[j]
                pt = ptv[j]
                pr = prv[j]
                u0 = hrows[e, pl.ds(ph, 16)] - trows[e, pl.ds(pt, 16)]
                u1 = hrows[e, pl.ds(ph + 16, 16)] - trows[e, pl.ds(pt + 16, 16)]
                u2 = hrows[e, pl.ds(ph + 32, 16)] - trows[e, pl.ds(pt + 32, 16)]
                u3 = hrows[e, pl.ds(ph + 48, 16)] - trows[e, pl.ds(pt + 48, 16)]
                w0 = wrows[e, pl.ds(pr, 16)]
                w1 = wrows[e, pl.ds(pr + 16, 16)]
                w2 = wrows[e, pl.ds(pr + 32, 16)]
                w3 = wrows[e, pl.ds(pr + 48, 16)]
                m = (u0 * w0 + u1 * w1) + (u2 * w2 + u3 * w3)
                a = jnp.sum(m)
                x0 = u0 + rrows[e, pl.ds(pr, 16)] - a * w0
                x1 = u1 + rrows[e, pl.ds(pr + 16, 16)] - a * w1
                x2 = u2 + rrows[e, pl.ds(pr + 32, 16)] - a * w2
                x3 = u3 + rrows[e, pl.ds(pr + 48, 16)] - a * w3
                s = (jnp.abs(x0) + jnp.abs(x1)) + (jnp.abs(x2) + jnp.abs(x3))
                acc = jnp.where(lane == j, jnp.sum(s), acc)
            oscr[pl.ds(off + g * 16, 16)] = acc
            return carry

        lax.fori_loop(0, C // 16, group, None)
        return carry0

    lax.fori_loop(0, BPW // C, chunk, None)

    pltpu.sync_copy(oscr, out_hbm.at[pl.ds(base, BPW)])


def kernel(head, relation, tail, entity_table, relation_table, normal_table):
    mesh = plsc.VectorSubcoreMesh(core_axis_name="c", subcore_axis_name="s")
    k = functools.partial(
        pl.kernel,
        mesh=mesh,
        compiler_params=pltpu.CompilerParams(needs_layout_passes=False),
        out_type=jax.ShapeDtypeStruct((B,), jnp.float32),
        scratch_types=[
            pltpu.VMEM((BPW,), jnp.int32),
            pltpu.VMEM((BPW,), jnp.int32),
            pltpu.VMEM((BPW,), jnp.int32),
            pltpu.VMEM((BPW,), jnp.int32),
            pltpu.VMEM((BPW,), jnp.int32),
            pltpu.VMEM((BPW,), jnp.int32),
            pltpu.VMEM((C, 128), jnp.float32),
            pltpu.VMEM((C, 128), jnp.float32),
            pltpu.VMEM((C, 128), jnp.float32),
            pltpu.VMEM((C, 128), jnp.float32),
            pltpu.VMEM((BPW,), jnp.float32),
            pltpu.SemaphoreType.DMA,
        ],
    )(_tec_body)
    ent2 = _repack_entity_table(entity_table.T)
    relt2 = jnp.reshape(relation_table, (500, 128))
    nrm2 = jnp.reshape(normal_table, (500, 128))
    return k(head, relation, tail, ent2, relt2, nrm2)


# speedup vs baseline: 3.7171x; 3.7171x over previous
import functools

import jax
import jax.numpy as jnp
from jax import lax
from jax.experimental import pallas as pl
from jax.experimental.pallas import tpu as pltpu
from jax.experimental.pallas import tpu_sc as plsc

B = 16384
D = 64
NC = 2
NS = 16
NW = NC * NS
BPW = B // NW
C = 16
NCH = BPW // C


def _tec_body(head_hbm, rel_hbm, tail_hbm, ent_hbm, relt_hbm, nrm_hbm,
              out_hbm, hidx, tidx, ridx, roff, hstages, tstages, rrowss,
              wrowss, oscr, sems):
    wid = lax.axis_index("s") * NC + lax.axis_index("c")
    base = wid * BPW

    pltpu.sync_copy(head_hbm.at[pl.ds(base, BPW)], hidx)
    pltpu.sync_copy(tail_hbm.at[pl.ds(base, BPW)], tidx)
    pltpu.sync_copy(rel_hbm.at[pl.ds(base, BPW)], ridx)

    def split(g, _):
        rv = ridx[pl.ds(g * 16, 16)]
        roff[pl.ds(g * 16, 16)] = (rv & 1) << 6
        ridx[pl.ds(g * 16, 16)] = rv >> 1
        return _

    lax.fori_loop(0, BPW // 16, split, None)

    lane = lax.iota(jnp.int32, 16)

    def fire(c, b):
        c = jnp.minimum(c, NCH - 1)
        off = c * C

        def fg(g, _):
            hv = hidx[pl.ds(off + g * 16, 16)]
            tv = tidx[pl.ds(off + g * 16, 16)]
            for j in range(16):
                el = g * 16 + j
                hs = pl.multiple_of((hv[j] >> 3) * 8, 8)
                ts = pl.multiple_of((tv[j] >> 3) * 8, 8)
                pltpu.async_copy(ent_hbm.at[pl.ds(hs, 8), :],
                                 hstages[b].at[pl.ds(el * 8, 8), :],
                                 sems[b])
                pltpu.async_copy(ent_hbm.at[pl.ds(ts, 8), :],
                                 tstages[b].at[pl.ds(el * 8, 8), :],
                                 sems[b])
            return _

        lax.fori_loop(0, C // 16, fg, None)
        pltpu.async_copy(relt_hbm.at[ridx.at[pl.ds(off, C)]], rrowss[b],
                         sems[b])
        pltpu.async_copy(nrm_hbm.at[ridx.at[pl.ds(off, C)]], wrowss[b],
                         sems[b])

    def drain(b):
        pltpu.make_async_copy(ent_hbm.at[pl.ds(0, C * 8), :], hstages[b],
                              sems[b]).wait()
        pltpu.make_async_copy(ent_hbm.at[pl.ds(0, C * 8), :], tstages[b],
                              sems[b]).wait()
        pltpu.make_async_copy(relt_hbm.at[pl.ds(0, C), :], rrowss[b],
                              sems[b]).wait()
        pltpu.make_async_copy(relt_hbm.at[pl.ds(0, C), :], wrowss[b],
                              sems[b]).wait()

    fire(0, 0)

    def chunk2(h, carry0):
        for b in range(2):
            c = h * 2 + b
            off = c * C
            fire(c + 1, 1 - b)
            drain(b)
            hstage = hstages[b]
            tstage = tstages[b]
            rrows = rrowss[b]
            wrows = wrowss[b]

            def group(g, carry):
                acc = jnp.zeros((16,), jnp.float32)
                prv = roff[pl.ds(off + g * 16, 16)]
                hv = hidx[pl.ds(off + g * 16, 16)]
                tv = tidx[pl.ds(off + g * 16, 16)]
                for j in range(16):
                    e = g * 16 + j
                    pr = prv[j]
                    hr = e * 8 + (hv[j] & 7)
                    tr = e * 8 + (tv[j] & 7)
                    u0 = hstage[hr, pl.ds(0, 16)] - tstage[tr, pl.ds(0, 16)]
                    u1 = hstage[hr, pl.ds(16, 16)] - tstage[tr, pl.ds(16, 16)]
                    u2 = hstage[hr, pl.ds(32, 16)] - tstage[tr, pl.ds(32, 16)]
                    u3 = hstage[hr, pl.ds(48, 16)] - tstage[tr, pl.ds(48, 16)]
                    w0 = wrows[e, pl.ds(pr, 16)]
                    w1 = wrows[e, pl.ds(pr + 16, 16)]
                    w2 = wrows[e, pl.ds(pr + 32, 16)]
                    w3 = wrows[e, pl.ds(pr + 48, 16)]
                    m = (u0 * w0 + u1 * w1) + (u2 * w2 + u3 * w3)
                    a = jnp.sum(m)
                    x0 = u0 + rrows[e, pl.ds(pr, 16)] - a * w0
                    x1 = u1 + rrows[e, pl.ds(pr + 16, 16)] - a * w1
                    x2 = u2 + rrows[e, pl.ds(pr + 32, 16)] - a * w2
                    x3 = u3 + rrows[e, pl.ds(pr + 48, 16)] - a * w3
                    s = (jnp.abs(x0) + jnp.abs(x1)) + (jnp.abs(x2)
                                                       + jnp.abs(x3))
                    acc = jnp.where(lane == j, jnp.sum(s), acc)
                oscr[pl.ds(off + g * 16, 16)] = acc
                return carry

            lax.fori_loop(0, C // 16, group, None)
        return carry0

    lax.fori_loop(0, NCH // 2, chunk2, None)
    drain(0)

    pltpu.sync_copy(oscr, out_hbm.at[pl.ds(base, BPW)])


def kernel(head, relation, tail, entity_table, relation_table, normal_table):
    mesh = plsc.VectorSubcoreMesh(core_axis_name="c", subcore_axis_name="s")
    k = functools.partial(
        pl.kernel,
        mesh=mesh,
        compiler_params=pltpu.CompilerParams(needs_layout_passes=False),
        out_type=jax.ShapeDtypeStruct((B,), jnp.float32),
        scratch_types=[
            pltpu.VMEM((BPW,), jnp.int32),
            pltpu.VMEM((BPW,), jnp.int32),
            pltpu.VMEM((BPW,), jnp.int32),
            pltpu.VMEM((BPW,), jnp.int32),
            [pltpu.VMEM((C * 8, D), jnp.float32) for _ in range(2)],
            [pltpu.VMEM((C * 8, D), jnp.float32) for _ in range(2)],
            [pltpu.VMEM((C, 128), jnp.float32) for _ in range(2)],
            [pltpu.VMEM((C, 128), jnp.float32) for _ in range(2)],
            pltpu.VMEM((BPW,), jnp.float32),
            [pltpu.SemaphoreType.DMA for _ in range(2)],
        ],
    )(_tec_body)
    relt2 = jnp.reshape(relation_table, (500, 128))
    nrm2 = jnp.reshape(normal_table, (500, 128))
    return k(head, relation, tail, entity_table, relt2, nrm2)
